# baseline (speedup 1.0000x reference)
import jax
import jax.numpy as jnp
from jax import lax
from jax.experimental import pallas as pl
from jax.experimental.pallas import tpu as pltpu
from jax.experimental.pallas import tpu_sc as plsc

N_VOCAB = 100000
N_EMBD = 64
N_TOKEN = 200
BATCH = 4096

NC = 2
NS = 16
NW = NC * NS
BBLK = BATCH // NW
LANES = 16
VPR = N_EMBD // LANES
TROWS = N_TOKEN * N_EMBD // 8
NBUF = 4


def _emb_kernel(table_hbm, idx_hbm, pos_hbm, out_hbm,
                idx_v, pos_v, bufs, buf2s, gsems, wsems):
    wid = lax.axis_index("s") * NC + lax.axis_index("c")
    col0 = wid * BBLK

    pltpu.sync_copy(idx_hbm.at[:, pl.ds(col0, BBLK)], idx_v)
    pltpu.sync_copy(pos_hbm, pos_v)

    def gather(t, b):
        return pltpu.make_async_copy(
            table_hbm.at[idx_v.at[t]], bufs[b], gsems[b])

    def write(t, b):
        return pltpu.make_async_copy(
            buf2s[b], out_hbm.at[pl.ds(8 * t, 8), wid], wsems[b])

    for b in range(NBUF):
        gather(b, b).start()

    iota = lax.iota(jnp.int32, LANES)

    def outer(k, carry):
        i = k * NBUF
        for b in range(NBUF):
            t = i + b
            gather(t, b).wait()

            @pl.when(t >= NBUF)
            def _():
                write(t - NBUF, b).wait()

            pv = [pos_v[t, pl.ds(c * LANES, LANES)] for c in range(VPR)]

            def tr_body(blk, c2):
                rows = blk * LANES + iota
                for d in range(LANES):
                    perm = (d + iota) & (LANES - 1)
                    r = perm & 7
                    p3 = perm >> 3
                    for c in range(VPR):
                        ecol = c * LANES + perm
                        pvr = pv[c].at[perm].get(mode="promise_in_bounds")
                        v = plsc.load_gather(bufs[b], [rows, ecol]) + pvr
                        plsc.store_scatter(
                            buf2s[b], [2 * c + p3, r, rows], v)
                return c2
            lax.fori_loop(0, BBLK // LANES, tr_body, 0, unroll=2)

            write(t, b).start()

            @pl.when(t + NBUF < N_TOKEN)
            def _():
                gather(t + NBUF, b).start()
        return carry

    lax.fori_loop(0, N_TOKEN // NBUF, outer, 0)

    for b in range(NBUF):
        write(N_TOKEN - NBUF + b, b).wait()


def _emb(table, idx_t, pos):
    mesh = plsc.VectorSubcoreMesh(core_axis_name="c", subcore_axis_name="s")
    f = pl.kernel(
        _emb_kernel,
        out_type=jax.ShapeDtypeStruct((TROWS, NW, 8, 128), jnp.float32),
        mesh=mesh,
        scratch_types=[
            pltpu.VMEM((N_TOKEN, BBLK), jnp.int32),
            pltpu.VMEM((N_TOKEN, N_EMBD), jnp.float32),
            [pltpu.VMEM((BBLK, N_EMBD), jnp.float32) for _ in range(NBUF)],
            [pltpu.VMEM((8, 8, BBLK), jnp.float32) for _ in range(NBUF)],
            [pltpu.SemaphoreType.DMA for _ in range(NBUF)],
            [pltpu.SemaphoreType.DMA for _ in range(NBUF)],
        ],
        compiler_params=pltpu.CompilerParams(
            use_tc_tiling_on_sc=False, needs_layout_passes=False),
    )
    return f(table, idx_t, pos)


def kernel(tokens, token_embedding, position_embedding):
    idx_t = tokens.T
    out4 = _emb(token_embedding, idx_t, position_embedding)
    y = out4.reshape(N_TOKEN, 8, NW, 8, 128).transpose(2, 4, 0, 1, 3)
    return y.reshape(BATCH, N_TOKEN, N_EMBD)


# speedup vs baseline: 1.0611x; 1.0611x over previous
import jax
import jax.numpy as jnp
from jax import lax
from jax.experimental import pallas as pl
from jax.experimental.pallas import tpu as pltpu
from jax.experimental.pallas import tpu_sc as plsc

N_VOCAB = 100000
N_EMBD = 64
N_TOKEN = 200
BATCH = 4096

NC = 2
NS = 16
NW = NC * NS
BBLK = BATCH // NW
LANES = 16
VPR = N_EMBD // LANES
TROWS = N_TOKEN * N_EMBD // 8
NBUF = 4


def _emb_kernel(table_hbm, idx_hbm, pos_hbm, out_hbm,
                idx_v, pos_v, bufs, buf2s, gsems, wsems):
    wid = lax.axis_index("s") * NC + lax.axis_index("c")
    col0 = wid * BBLK

    pltpu.sync_copy(idx_hbm.at[:, pl.ds(col0, BBLK)], idx_v)
    pltpu.sync_copy(pos_hbm, pos_v)

    def gather(t, b):
        return pltpu.make_async_copy(
            table_hbm.at[idx_v.at[t]], bufs[b], gsems[b])

    def write(t, b):
        return pltpu.make_async_copy(
            buf2s[b], out_hbm.at[pl.ds(8 * t, 8), wid], wsems[b])

    for b in range(NBUF):
        gather(b, b).start()

    iota = lax.iota(jnp.int32, LANES)

    def outer(k, carry):
        i = k * NBUF
        for b in range(NBUF):
            t = i + b
            gather(t, b).wait()

            @pl.when(t >= NBUF)
            def _():
                write(t - NBUF, b).wait()

            pv = [pos_v[t, pl.ds(c * LANES, LANES)] for c in range(VPR)]

            def tr_body(blk, c2):
                rows = blk * LANES + iota
                for d in range(LANES):
                    perm = (d + iota) & (LANES - 1)
                    r = perm & 7
                    p3 = perm >> 3
                    for c in range(VPR):
                        ecol = c * LANES + perm
                        pvr = pv[c].at[perm].get(mode="promise_in_bounds")
                        v = plsc.load_gather(bufs[b], [rows, ecol]) + pvr
                        plsc.store_scatter(
                            buf2s[b], [2 * c + p3, r, rows], v)
                return c2
            lax.fori_loop(0, BBLK // LANES, tr_body, 0)

            write(t, b).start()

            @pl.when(t + NBUF < N_TOKEN)
            def _():
                gather(t + NBUF, b).start()
        return carry

    lax.fori_loop(0, N_TOKEN // NBUF, outer, 0)

    for b in range(NBUF):
        write(N_TOKEN - NBUF + b, b).wait()


def _emb(table, idx_t, pos):
    mesh = plsc.VectorSubcoreMesh(core_axis_name="c", subcore_axis_name="s")
    f = pl.kernel(
        _emb_kernel,
        out_type=jax.ShapeDtypeStruct((TROWS, NW, 8, 128), jnp.float32),
        mesh=mesh,
        scratch_types=[
            pltpu.VMEM((N_TOKEN, BBLK), jnp.int32),
            pltpu.VMEM((N_TOKEN, N_EMBD), jnp.float32),
            [pltpu.VMEM((BBLK, N_EMBD), jnp.float32) for _ in range(NBUF)],
            [pltpu.VMEM((8, 8, BBLK), jnp.float32) for _ in range(NBUF)],
            [pltpu.SemaphoreType.DMA for _ in range(NBUF)],
            [pltpu.SemaphoreType.DMA for _ in range(NBUF)],
        ],
        compiler_params=pltpu.CompilerParams(
            use_tc_tiling_on_sc=False, needs_layout_passes=False),
    )
    return f(table, idx_t, pos)


def kernel(tokens, token_embedding, position_embedding):
    idx_t = tokens.T
    out4 = _emb(token_embedding, idx_t, position_embedding)
    y = out4.reshape(N_TOKEN, 8, NW, 8, 128).transpose(2, 4, 0, 1, 3)
    return y.reshape(BATCH, N_TOKEN, N_EMBD)


# speedup vs baseline: 1.1340x; 1.0687x over previous
import jax
import jax.numpy as jnp
from jax import lax
from jax.experimental import pallas as pl
from jax.experimental.pallas import tpu as pltpu
from jax.experimental.pallas import tpu_sc as plsc

N_VOCAB = 100000
N_EMBD = 64
N_TOKEN = 200
BATCH = 4096

NC = 2
NS = 16
NW = NC * NS
BBLK = BATCH // NW
LANES = 16
VPR = N_EMBD // LANES
TROWS = N_TOKEN * N_EMBD // 8
NBUF = 4


def _emb_kernel(table_hbm, idx_hbm, pos_hbm, out_hbm,
                idx_v, pos_v, bufs, buf2s, gsems, wsems):
    wid = lax.axis_index("s") * NC + lax.axis_index("c")
    col0 = wid * BBLK

    pltpu.sync_copy(idx_hbm.at[:, pl.ds(col0, BBLK)], idx_v)
    pltpu.sync_copy(pos_hbm, pos_v)

    def gather(t, b):
        return pltpu.make_async_copy(
            table_hbm.at[idx_v.at[t]], bufs[b], gsems[b])

    def write(t, b):
        return pltpu.make_async_copy(
            buf2s[b], out_hbm.at[pl.ds(8 * t, 8), wid], wsems[b])

    for b in range(NBUF):
        gather(b, b).start()

    iota = lax.iota(jnp.int32, LANES)

    def outer(k, carry):
        i = k * NBUF
        for b in range(NBUF):
            t = i + b
            gather(t, b).wait()

            @pl.when(t >= NBUF)
            def _():
                write(t - NBUF, b).wait()

            pv = [pos_v[t, pl.ds(c * LANES, LANES)] for c in range(VPR)]

            def tr_body(blk, c2):
                rows = blk * LANES + iota
                for c in range(VPR):
                    for d in range(LANES):
                        perm = (d + iota) & (LANES - 1)
                        ecol = c * LANES + perm
                        pvr = pv[c].at[perm].get(mode="promise_in_bounds")
                        v = plsc.load_gather(bufs[b], [rows, ecol]) + pvr
                        plsc.store_scatter(
                            buf2s[b], [ecol >> 3, ecol & 7, rows], v)
                return c2
            lax.fori_loop(0, BBLK // LANES, tr_body, 0)

            write(t, b).start()

            @pl.when(t + NBUF < N_TOKEN)
            def _():
                gather(t + NBUF, b).start()
        return carry

    lax.fori_loop(0, N_TOKEN // NBUF, outer, 0)

    for b in range(NBUF):
        write(N_TOKEN - NBUF + b, b).wait()


def _emb(table, idx_t, pos):
    mesh = plsc.VectorSubcoreMesh(core_axis_name="c", subcore_axis_name="s")
    f = pl.kernel(
        _emb_kernel,
        out_type=jax.ShapeDtypeStruct((TROWS, NW, 8, 128), jnp.float32),
        mesh=mesh,
        scratch_types=[
            pltpu.VMEM((N_TOKEN, BBLK), jnp.int32),
            pltpu.VMEM((N_TOKEN, N_EMBD), jnp.float32),
            [pltpu.VMEM((BBLK, N_EMBD), jnp.float32) for _ in range(NBUF)],
            [pltpu.VMEM((8, 8, BBLK), jnp.float32) for _ in range(NBUF)],
            [pltpu.SemaphoreType.DMA for _ in range(NBUF)],
            [pltpu.SemaphoreType.DMA for _ in range(NBUF)],
        ],
        compiler_params=pltpu.CompilerParams(
            use_tc_tiling_on_sc=False, needs_layout_passes=False),
    )
    return f(table, idx_t, pos)


def kernel(tokens, token_embedding, position_embedding):
    idx_t = tokens.T
    out4 = _emb(token_embedding, idx_t, position_embedding)
    y = out4.reshape(N_TOKEN, 8, NW, 8, 128).transpose(2, 4, 0, 1, 3)
    return y.reshape(BATCH, N_TOKEN, N_EMBD)
